# baseline (speedup 1.0000x reference)
import functools

import jax
import jax.numpy as jnp
from jax.experimental import pallas as pl
from jax.experimental.pallas import tpu as pltpu

R_ACT, R_BKG = 8, 8
CT = 512
SIGN = -2147483648


def _conv_stage(x_ref, w_ref, b_ref, f_ref):
    C = w_ref.shape[2]
    xb = x_ref[0]
    m0 = jnp.dot(xb, w_ref[0], preferred_element_type=jnp.float32)
    m1 = jnp.dot(xb, w_ref[1], preferred_element_type=jnp.float32)
    m2 = jnp.dot(xb, w_ref[2], preferred_element_type=jnp.float32)
    z = jnp.zeros((1, C), jnp.float32)
    conv = m1 + jnp.concatenate([z, m0[:-1]], axis=0) \
              + jnp.concatenate([m2[1:], z], axis=0)
    f_ref[0] = jnp.maximum(conv + b_ref[...], 0.0)


def _tdot(a, b, contract_a=1):
    return jax.lax.dot_general(
        a, b, (((contract_a,), (0,)), ((), ())),
        preferred_element_type=jnp.float32)


def _select_stage(f_ref, cls_ref,
                  sa_ref, sb_ref, fa_ref, fb_ref, cso_ref):
    T = f_ref.shape[1]
    NCLS = cls_ref.shape[1]
    K = T // R_ACT

    feats = f_ref[0]
    cas = jnp.dot(feats, cls_ref[...], preferred_element_type=jnp.float32)

    cm = jnp.max(cas, axis=1, keepdims=True)
    e = jnp.exp(cas - cm)
    cso_ref[0] = e / jnp.sum(e, axis=1, keepdims=True)

    isub = jax.lax.broadcasted_iota(jnp.int32, (T, T), 0)
    ilan = jax.lax.broadcasted_iota(jnp.int32, (T, T), 1)
    ident = (isub == ilan).astype(jnp.float32)

    mcol = jnp.sqrt(jnp.sum(feats * feats, axis=1, keepdims=True))
    mrow = _tdot(mcol, ident, contract_a=0)

    offdiag = isub != ilan
    tie = (mcol == mrow) & (isub < ilan)
    beats_a = (((mcol > mrow) | tie) & offdiag).astype(jnp.int32)
    rank_a = jnp.sum(beats_a, axis=0, keepdims=True)
    beats_b = (((mcol < mrow) | tie) & offdiag).astype(jnp.int32)
    rank_b = jnp.sum(beats_b, axis=0, keepdims=True)

    iota_k = jax.lax.broadcasted_iota(jnp.int32, (K, T), 0)
    oh_a = (iota_k == rank_a).astype(jnp.float32)
    oh_b = (iota_k == rank_b).astype(jnp.float32)
    fa_ref[0] = _tdot(oh_a, feats)
    fb_ref[0] = _tdot(oh_b, feats)

    mask_b = (rank_b < K).astype(jnp.float32)
    sb = _tdot(mask_b, cas) / K
    eb = jnp.exp(sb - jnp.max(sb))
    sb_ref[0] = eb / jnp.sum(eb)

    casT = _tdot(cas, ident, contract_a=0)
    bits = jax.lax.bitcast_convert_type(casT, jnp.int32)
    sgn = jnp.int32(SIGN)
    keys_s = jnp.where(bits < 0, ~bits, bits ^ sgn) ^ sgn
    prefix = jnp.zeros((NCLS, 1), jnp.int32)
    for bit in range(31, -1, -1):
        bval = sgn if bit == 31 else jnp.int32(1 << bit)
        cand = prefix | bval
        cand_s = cand ^ sgn
        cnt = jnp.sum((keys_s >= cand_s).astype(jnp.int32), axis=1,
                      keepdims=True)
        prefix = jnp.where(cnt >= K, cand, prefix)
    theta_s = prefix ^ sgn
    tbits = jnp.where(prefix < 0, prefix ^ sgn, ~prefix)
    theta = jax.lax.bitcast_convert_type(tbits, jnp.float32)
    gt = keys_s > theta_s
    sum_gt = jnp.sum(jnp.where(gt, casT, 0.0), axis=1, keepdims=True)
    cnt_gt = jnp.sum(gt.astype(jnp.int32), axis=1, keepdims=True)
    stk = sum_gt + (K - cnt_gt).astype(jnp.float32) * theta
    identc = (jax.lax.broadcasted_iota(jnp.int32, (NCLS, NCLS), 0) ==
              jax.lax.broadcasted_iota(jnp.int32, (NCLS, NCLS), 1)
              ).astype(jnp.float32)
    sa = _tdot(stk / K, identc, contract_a=0)
    ea = jnp.exp(sa - jnp.max(sa))
    sa_ref[0] = ea / jnp.sum(ea)


def kernel(x, conv_w, conv_b, cls_w):
    B, T, Fdim = x.shape
    C = conv_w.shape[0]
    NCLS = cls_w.shape[0]
    K = T // R_ACT
    ct = min(CT, C)
    NC = C // ct

    w3 = jnp.transpose(conv_w, (2, 1, 0))
    bias = conv_b.reshape(1, C)
    clsw = jnp.transpose(cls_w[:, :, 0], (1, 0))

    feats = pl.pallas_call(
        _conv_stage,
        grid=(NC, B),
        in_specs=[
            pl.BlockSpec((1, T, Fdim), lambda c, b: (b, 0, 0)),
            pl.BlockSpec((3, Fdim, ct), lambda c, b: (0, 0, c)),
            pl.BlockSpec((1, ct), lambda c, b: (0, c)),
        ],
        out_specs=pl.BlockSpec((1, T, ct), lambda c, b: (b, 0, c)),
        out_shape=jax.ShapeDtypeStruct((B, T, C), jnp.float32),
        compiler_params=pltpu.CompilerParams(
            dimension_semantics=("arbitrary", "arbitrary"),
        ),
    )(x, w3, bias)

    out_shapes = (
        jax.ShapeDtypeStruct((B, 1, NCLS), jnp.float32),
        jax.ShapeDtypeStruct((B, 1, NCLS), jnp.float32),
        jax.ShapeDtypeStruct((B, K, C), jnp.float32),
        jax.ShapeDtypeStruct((B, K, C), jnp.float32),
        jax.ShapeDtypeStruct((B, T, NCLS), jnp.float32),
    )
    sa, sb, fa, fb, cso = pl.pallas_call(
        _select_stage,
        grid=(B,),
        in_specs=[
            pl.BlockSpec((1, T, C), lambda b: (b, 0, 0)),
            pl.BlockSpec((C, NCLS), lambda b: (0, 0)),
        ],
        out_specs=(
            pl.BlockSpec((1, 1, NCLS), lambda b: (b, 0, 0)),
            pl.BlockSpec((1, 1, NCLS), lambda b: (b, 0, 0)),
            pl.BlockSpec((1, K, C), lambda b: (b, 0, 0)),
            pl.BlockSpec((1, K, C), lambda b: (b, 0, 0)),
            pl.BlockSpec((1, T, NCLS), lambda b: (b, 0, 0)),
        ),
        out_shape=out_shapes,
        compiler_params=pltpu.CompilerParams(
            dimension_semantics=("arbitrary",),
        ),
    )(feats, clsw)
    return (sa[:, 0, :], sb[:, 0, :], fa, fb, feats, cso)
